# baseline (speedup 1.0000x reference)
import functools
import jax
import jax.numpy as jnp
from jax import lax
from jax.experimental import pallas as pl
from jax.experimental.pallas import tpu as pltpu
from jax.experimental.pallas import tpu_sc as plsc

N = 2048
H = 128
NV = N // 16
GAMMA = 0.5


def _sc_stage1(uv_hbm, aux_hbm, a_hbm, s_hbm,
               arows_out, srows_out, m2_out,
               uv_v, idx0_v, gath_v, arow_v, myidx_v, row_v, acc_v,
               sh_row, sh_m2):
    c = lax.axis_index("c")
    s = lax.axis_index("s")

    pltpu.sync_copy(aux_hbm, idx0_v)

    @pl.when(s == 0)
    def _leader():
        pltpu.sync_copy(uv_hbm, uv_v)
        pltpu.sync_copy(a_hbm.at[uv_v], gath_v)
        pltpu.sync_copy(gath_v.at[pl.ds(c, 1)], sh_row)

        def _z(v, carry):
            acc_v[0, pl.ds(v * 16, 16)] = jnp.zeros((16,), jnp.float32)
            return carry
        lax.fori_loop(0, NV, _z, 0)
        pltpu.sync_copy(acc_v, sh_m2)

        @pl.when(c == 0)
        def _io():
            pltpu.sync_copy(gath_v, arows_out)
            pltpu.sync_copy(s_hbm.at[uv_v], gath_v)
            pltpu.sync_copy(gath_v, srows_out)

    plsc.subcore_barrier()

    pltpu.sync_copy(sh_row, arow_v)

    def _zidx(v, carry):
        myidx_v[pl.ds(v * 16, 16)] = jnp.zeros((16,), jnp.int32)
        return carry
    lax.fori_loop(0, 64, _zidx, 0)

    lane = jnp.arange(16, dtype=jnp.int32)

    def _compact(v, base):
        vec = arow_v[0, pl.ds(v * 16, 16)]
        m = vec > 0.0
        p = base + plsc.cumsum(m.astype(jnp.int32)) - 1
        keep = jnp.logical_and(m, (p % 16) == s)
        pos = 8 * (p // 16)
        cols = lane + v * 16
        plsc.store_scatter(myidx_v, [pos], cols, mask=keep)
        return base + plsc.all_reduce_population_count(m)

    base = lax.fori_loop(0, NV, _compact, jnp.zeros((16,), jnp.int32))
    n = jnp.max(base)
    nw = jnp.where(n > s, (n - s + 15) // 16, 0)

    def _zero(v, carry):
        acc_v[0, pl.ds(v * 16, 16)] = jnp.zeros((16,), jnp.float32)
        return carry
    lax.fori_loop(0, NV, _zero, 0)

    def _cond(j):
        return j < nw

    def _body(j):
        pltpu.sync_copy(a_hbm.at[myidx_v.at[pl.ds(8 * j, 1)]], row_v)

        def _acc(v, carry):
            sl = pl.ds(v * 16, 16)
            acc_v[0, sl] = acc_v[0, sl] + row_v[0, sl]
            return carry
        lax.fori_loop(0, NV, _acc, 0)
        return j + 1

    lax.while_loop(_cond, _body, jnp.int32(0))

    @pl.when(nw > 0)
    def _combine():
        pltpu.sync_copy(acc_v, sh_m2.at[idx0_v], add=True)

    plsc.subcore_barrier()

    @pl.when(s == 0)
    def _writeout():
        pltpu.sync_copy(sh_m2, m2_out.at[pl.ds(c, 1)])


def _sc_call(uv, A, S):
    aux = jnp.zeros((1,), jnp.int32)
    mesh = plsc.VectorSubcoreMesh(core_axis_name="c", subcore_axis_name="s")
    k = functools.partial(
        pl.kernel,
        mesh=mesh,
        out_type=[
            jax.ShapeDtypeStruct((2, N), jnp.float32),
            jax.ShapeDtypeStruct((2, N), jnp.float32),
            jax.ShapeDtypeStruct((2, N), jnp.float32),
        ],
        scratch_types=[
            pltpu.VMEM((2,), jnp.int32),
            pltpu.VMEM((1,), jnp.int32),
            pltpu.VMEM((2, N), jnp.float32),
            pltpu.VMEM((1, N), jnp.float32),
            pltpu.VMEM((1024,), jnp.int32),
            pltpu.VMEM((1, N), jnp.float32),
            pltpu.VMEM((1, N), jnp.float32),
            pltpu.VMEM_SHARED((1, N), jnp.float32),
            pltpu.VMEM_SHARED((1, N), jnp.float32),
        ],
        compiler_params=pltpu.CompilerParams(needs_layout_passes=False),
    )(_sc_stage1)
    return k(uv, aux, A, S)


def _rt(x, w):
    return lax.dot_general(x, w, (((1,), (1,)), ((), ())),
                           preferred_element_type=jnp.float32)


def _final_kernel(uv_ref, pe_ref, arows_ref, srows_ref, m2_ref,
                  wh_ref, whb_ref, ws_ref, wsb_ref, wr_ref, wrb_ref,
                  wt_ref, wtb_ref, sim_ref, td_ref, out_ref):
    pe = pe_ref[...]
    h_prev = _rt(pe, wh_ref[...]) + whb_ref[...]
    six = jnp.concatenate(
        [arows_ref[...], m2_ref[...], srows_ref[...]], axis=0)
    sixT = six.T
    a_col = sixT[:, 0:2]
    m2_col = sixT[:, 2:4]
    s_col = sixT[:, 4:6]
    mask = jnp.logical_or(a_col > 0, m2_col > 0)
    base = (1.0 - GAMMA) * sim_ref[0, 0]
    q = jnp.where(mask, jnp.exp(base + GAMMA * s_col), 0.0)
    qs = jnp.sum(q, axis=0, keepdims=True) + 1e-7
    qn = q / qs
    nn = jnp.sum(mask.astype(jnp.float32), axis=0, keepdims=True)
    hs = []
    for c in (0, 1):
        cc = 1 - c
        x = qn[:, cc:cc + 1] * h_prev
        x = jnp.where(mask[:, cc:cc + 1], x, -1e30)
        m = jnp.max(x, axis=0, keepdims=True)
        h = jax.nn.sigmoid(m)
        h = jnp.where(nn[0, cc] > 0, h, jnp.zeros_like(h))
        hs.append(h)
    h_struct = jnp.concatenate(hs, axis=0)
    h1 = _rt(h_struct, ws_ref[...]) + wsb_ref[...]
    u1 = uv_ref[0]
    u2 = uv_ref[1]
    pe_rows = jnp.concatenate(
        [pe_ref[pl.ds(u1, 1), :], pe_ref[pl.ds(u2, 1), :]], axis=0)
    h2 = _rt(pe_rows, wr_ref[...]) + wrb_ref[...]
    h3 = _rt(td_ref[...], wt_ref[...]) + wtb_ref[...]
    z = jax.nn.sigmoid(h1 + h2 + h3)
    out_ref[...] = pe
    out_ref[pl.ds(u1, 1), :] = z[0:1, :]
    out_ref[pl.ds(u2, 1), :] = z[1:2, :]


def _pin(shape):
    return pl.BlockSpec(shape, lambda i, uv, n=len(shape): (0,) * n)


def kernel(prev_embed, A, S, W_h_w, W_h_b, W_struct_w, W_struct_b,
           W_rec_w, W_rec_b, W_t_w, W_t_b, sim, time_delta_uv, node1, node2):
    uv = jnp.stack([jnp.asarray(node1, jnp.int32),
                    jnp.asarray(node2, jnp.int32)])
    arows, srows, m2 = _sc_call(uv, A, S)

    whb = W_h_b.reshape(1, H)
    wsb = W_struct_b.reshape(1, H)
    wrb = W_rec_b.reshape(1, H)
    wtb = W_t_b.reshape(1, H)
    sim1 = jnp.reshape(sim, (1, 1)).astype(jnp.float32)

    grid = pltpu.PrefetchScalarGridSpec(
        num_scalar_prefetch=1,
        grid=(1,),
        in_specs=[
            _pin((N, H)),
            _pin((2, N)),
            _pin((2, N)),
            _pin((2, N)),
            _pin((H, H)),
            _pin((1, H)),
            _pin((H, H)),
            _pin((1, H)),
            _pin((H, H)),
            _pin((1, H)),
            _pin((H, 4)),
            _pin((1, H)),
            _pin((1, 1)),
            _pin((2, 4)),
        ],
        out_specs=[_pin((N, H))],
    )
    (z_new,) = pl.pallas_call(
        _final_kernel,
        grid_spec=grid,
        out_shape=[jax.ShapeDtypeStruct((N, H), jnp.float32)],
    )(uv, prev_embed, arows, srows, m2,
      W_h_w, whb, W_struct_w, wsb, W_rec_w, wrb, W_t_w, wtb,
      sim1, time_delta_uv)
    return z_new


# speedup vs baseline: 1.0090x; 1.0090x over previous
import functools
import jax
import jax.numpy as jnp
from jax import lax
from jax.experimental import pallas as pl
from jax.experimental.pallas import tpu as pltpu
from jax.experimental.pallas import tpu_sc as plsc

N = 2048
H = 128
NV = N // 16
GAMMA = 0.5


NCHUNK = NV // 16


def _sc_stage1(uv_hbm, aux_hbm, a_hbm,
               arows_out, m2_out,
               uv_v, idx0_v, gath_v, arow_v, myidx_v, row_v, acc_v,
               sh_row, sh_m2):
    c = lax.axis_index("c")
    s = lax.axis_index("s")

    pltpu.sync_copy(aux_hbm, idx0_v)

    def _zacc(v, carry):
        acc_v[0, pl.ds(v * 16, 16)] = jnp.zeros((16,), jnp.float32)
        return carry
    lax.fori_loop(0, NV, _zacc, 0)

    def _zidx(v, carry):
        myidx_v[pl.ds(v * 16, 16)] = jnp.zeros((16,), jnp.int32)
        return carry
    lax.fori_loop(0, 64, _zidx, 0)

    @pl.when(s == 0)
    def _leader():
        pltpu.sync_copy(uv_hbm, uv_v)
        pltpu.sync_copy(a_hbm.at[uv_v], gath_v)
        pltpu.sync_copy(gath_v.at[pl.ds(c, 1)], sh_row)
        pltpu.sync_copy(acc_v, sh_m2)

        @pl.when(c == 0)
        def _io():
            pltpu.sync_copy(gath_v, arows_out)

    plsc.subcore_barrier()

    pltpu.sync_copy(sh_row, arow_v)

    lane = jnp.arange(16, dtype=jnp.int32)

    def _compact(v, base):
        vec = arow_v[0, pl.ds(v * 16, 16)]
        m = vec > 0.0
        p = base + plsc.cumsum(m.astype(jnp.int32)) - 1
        pos = 8 * p
        cols = lane + v * 16
        plsc.store_scatter(myidx_v, [pos], cols, mask=m)
        return base + plsc.all_reduce_population_count(m)

    base = lax.fori_loop(s * NCHUNK, (s + 1) * NCHUNK, _compact,
                         jnp.zeros((16,), jnp.int32))
    nw = jnp.max(base)

    def _cond(j):
        return j < nw

    def _body(j):
        pltpu.sync_copy(a_hbm.at[myidx_v.at[pl.ds(8 * j, 1)]], row_v)

        def _acc(v, carry):
            sl = pl.ds(v * 16, 16)
            acc_v[0, sl] = acc_v[0, sl] + row_v[0, sl]
            return carry
        lax.fori_loop(0, NV, _acc, 0)
        return j + 1

    lax.while_loop(_cond, _body, jnp.int32(0))

    @pl.when(nw > 0)
    def _combine():
        pltpu.sync_copy(acc_v, sh_m2.at[idx0_v], add=True)

    plsc.subcore_barrier()

    @pl.when(s == 0)
    def _writeout():
        pltpu.sync_copy(sh_m2, m2_out.at[pl.ds(c, 1)])


def _sc_call(uv, A):
    aux = jnp.zeros((1,), jnp.int32)
    mesh = plsc.VectorSubcoreMesh(core_axis_name="c", subcore_axis_name="s")
    k = functools.partial(
        pl.kernel,
        mesh=mesh,
        out_type=[
            jax.ShapeDtypeStruct((2, N), jnp.float32),
            jax.ShapeDtypeStruct((2, N), jnp.float32),
        ],
        scratch_types=[
            pltpu.VMEM((2,), jnp.int32),
            pltpu.VMEM((1,), jnp.int32),
            pltpu.VMEM((2, N), jnp.float32),
            pltpu.VMEM((1, N), jnp.float32),
            pltpu.VMEM((1024,), jnp.int32),
            pltpu.VMEM((1, N), jnp.float32),
            pltpu.VMEM((1, N), jnp.float32),
            pltpu.VMEM_SHARED((1, N), jnp.float32),
            pltpu.VMEM_SHARED((1, N), jnp.float32),
        ],
        compiler_params=pltpu.CompilerParams(needs_layout_passes=False),
    )(_sc_stage1)
    return k(uv, aux, A)


def _rt(x, w):
    return lax.dot_general(x, w, (((1,), (1,)), ((), ())),
                           preferred_element_type=jnp.float32)


def _final_kernel(uv_ref, pe_ref, arows_ref, m2_ref,
                  wh_ref, whb_ref, ws_ref, wsb_ref, wr_ref, wrb_ref,
                  wt_ref, wtb_ref, sim_ref, td_ref, out_ref):
    pe = pe_ref[...]
    h_prev = _rt(pe, wh_ref[...]) + whb_ref[...]
    four = jnp.concatenate([arows_ref[...], m2_ref[...]], axis=0)
    fourT = four.T
    a_col = fourT[:, 0:2]
    m2_col = fourT[:, 2:4]
    deg = jnp.sum(a_col, axis=0, keepdims=True)
    s_col = a_col / jnp.maximum(deg, 1.0)
    mask = jnp.logical_or(a_col > 0, m2_col > 0)
    base = (1.0 - GAMMA) * sim_ref[0, 0]
    q = jnp.where(mask, jnp.exp(base + GAMMA * s_col), 0.0)
    qs = jnp.sum(q, axis=0, keepdims=True) + 1e-7
    qn = q / qs
    nn = jnp.sum(mask.astype(jnp.float32), axis=0, keepdims=True)
    hs = []
    for c in (0, 1):
        cc = 1 - c
        x = qn[:, cc:cc + 1] * h_prev
        x = jnp.where(mask[:, cc:cc + 1], x, -1e30)
        m = jnp.max(x, axis=0, keepdims=True)
        h = jax.nn.sigmoid(m)
        h = jnp.where(nn[0, cc] > 0, h, jnp.zeros_like(h))
        hs.append(h)
    h_struct = jnp.concatenate(hs, axis=0)
    h1 = _rt(h_struct, ws_ref[...]) + wsb_ref[...]
    u1 = uv_ref[0]
    u2 = uv_ref[1]
    pe_rows = jnp.concatenate(
        [pe_ref[pl.ds(u1, 1), :], pe_ref[pl.ds(u2, 1), :]], axis=0)
    h2 = _rt(pe_rows, wr_ref[...]) + wrb_ref[...]
    h3 = _rt(td_ref[...], wt_ref[...]) + wtb_ref[...]
    z = jax.nn.sigmoid(h1 + h2 + h3)
    out_ref[...] = pe
    out_ref[pl.ds(u1, 1), :] = z[0:1, :]
    out_ref[pl.ds(u2, 1), :] = z[1:2, :]


def _pin(shape):
    return pl.BlockSpec(shape, lambda i, uv, n=len(shape): (0,) * n)


def kernel(prev_embed, A, S, W_h_w, W_h_b, W_struct_w, W_struct_b,
           W_rec_w, W_rec_b, W_t_w, W_t_b, sim, time_delta_uv, node1, node2):
    uv = jnp.stack([jnp.asarray(node1, jnp.int32),
                    jnp.asarray(node2, jnp.int32)])
    arows, m2 = _sc_call(uv, A)

    whb = W_h_b.reshape(1, H)
    wsb = W_struct_b.reshape(1, H)
    wrb = W_rec_b.reshape(1, H)
    wtb = W_t_b.reshape(1, H)
    sim1 = jnp.reshape(sim, (1, 1)).astype(jnp.float32)

    grid = pltpu.PrefetchScalarGridSpec(
        num_scalar_prefetch=1,
        grid=(1,),
        in_specs=[
            _pin((N, H)),
            _pin((2, N)),
            _pin((2, N)),
            _pin((H, H)),
            _pin((1, H)),
            _pin((H, H)),
            _pin((1, H)),
            _pin((H, H)),
            _pin((1, H)),
            _pin((H, 4)),
            _pin((1, H)),
            _pin((1, 1)),
            _pin((2, 4)),
        ],
        out_specs=[_pin((N, H))],
    )
    (z_new,) = pl.pallas_call(
        _final_kernel,
        grid_spec=grid,
        out_shape=[jax.ShapeDtypeStruct((N, H), jnp.float32)],
    )(uv, prev_embed, arows, m2,
      W_h_w, whb, W_struct_w, wsb, W_rec_w, wrb, W_t_w, wtb,
      sim1, time_delta_uv)
    return z_new
